# Initial kernel scaffold; baseline (speedup 1.0000x reference)
import jax
import jax.numpy as jnp
from jax.experimental import pallas as pl


def kernel(x, edge_index, edge_weight, batch, W1, b1, W2, b2, Wih, Whh, bih, bhh, Wlin, blin):
    raise NotImplementedError("write your pallas kernel here")


# speedup vs baseline: 24.5261x; 24.5261x over previous
import functools

import jax
import jax.numpy as jnp
from jax import lax
from jax.experimental import pallas as pl
from jax.experimental.pallas import tpu as pltpu
from jax.experimental.pallas import tpu_sc as plsc

NN = 31744
EE = 1015808
FF = 16
TT = 8
HH = 256
BBG = 512
NOUT = 8
NCHG = 62

NSUB = 16
NSLAB = 4
CSLAB = 32
RPT = NN // NSUB
EPT = EE // NSUB
ECK = 512
NCK = EPT // ECK
EPW = EE // 32
NUCK = 31
UCKR = 8

def _newton_rsqrt(x):
    bits = lax.bitcast_convert_type(x, jnp.int32)
    y = lax.bitcast_convert_type(jnp.int32(0x5F3759DF) - (bits >> 1),
                                 jnp.float32)
    for _ in range(4):
        y = y * (1.5 - 0.5 * x * y * y)
    return y


def _edge_weight_body(row_u, col_u, ew_u, w_out,
                      idxv, valv, deg_sh, disv, rv, cv, ev, wv, zv):
    c = lax.axis_index("c")
    s = lax.axis_index("s")
    wid = c * NSUB + s

    zero16 = jnp.zeros((16,), jnp.float32)

    def zfill(i, _):
        zv[pl.ds(i * 16, 16)] = zero16
        return 0
    lax.fori_loop(0, 496 // 16, zfill, 0)

    def zdeg(i, _):
        pltpu.sync_copy(zv, deg_sh.at[pl.ds(s * RPT + i * 496, 496)])
        return 0
    lax.fori_loop(0, RPT // 496, zdeg, 0)
    plsc.subcore_barrier()

    def dchunk(k, _):
        stripe = 2 * s + k // NUCK
        ck = k % NUCK
        pltpu.sync_copy(row_u.at[stripe, ck], idxv)
        pltpu.sync_copy(ew_u.at[stripe, ck], valv)
        for j in range(UCKR):
            pltpu.sync_copy(valv.at[j], deg_sh.at[idxv.at[j]], add=True)
        return 0
    lax.fori_loop(0, 2 * NUCK, dchunk, 0)
    plsc.subcore_barrier()

    pltpu.sync_copy(deg_sh, disv)

    def rbody(i, _):
        d = disv[pl.ds(i * 16, 16)]
        y = _newton_rsqrt(jnp.maximum(d, 1e-30))
        disv[pl.ds(i * 16, 16)] = jnp.where(d > 0, y, 0.0)
        return 0
    lax.fori_loop(0, NN // 16, rbody, 0)

    def wchunk(k, _):
        pltpu.sync_copy(row_u.at[wid, k], rv)
        pltpu.sync_copy(col_u.at[wid, k], cv)
        pltpu.sync_copy(ew_u.at[wid, k], ev)
        for j in range(UCKR):
            def wbody(i, _):
                sl = pl.ds(i * 16, 16)
                dr = plsc.load_gather(disv, [rv[j, sl]])
                dc = plsc.load_gather(disv, [cv[j, sl]])
                wv[j, sl] = -(dr * ev[j, sl] * dc)
                return 0
            lax.fori_loop(0, 8, wbody, 0)
        pltpu.sync_copy(wv, w_out.at[wid, k])
        return 0
    lax.fori_loop(0, NUCK, wchunk, 0)


def _spmm_body(xs, row_s, col_s, w_s, out, riv, civ, wv, gbuf, zbuf, acc, sem):
    c = lax.axis_index("c")
    s = lax.axis_index("s")

    zero16 = jnp.zeros((16,), jnp.float32)

    def zfill(i, _):
        zbuf[i, pl.ds(0, 16)] = zero16
        zbuf[i, pl.ds(16, 16)] = zero16
        return 0
    lax.fori_loop(0, 496, zfill, 0)

    for slab_i in range(2):
        slab = c * 2 + slab_i

        def zc(i, _):
            pltpu.sync_copy(zbuf, acc.at[pl.ds(s * RPT + i * 496, 496)])
            return 0
        lax.fori_loop(0, RPT // 496, zc, 0)
        plsc.subcore_barrier()

        def echunk(k, _):
            pltpu.sync_copy(row_s.at[s, k], riv)
            pltpu.sync_copy(col_s.at[s, k], civ)
            pltpu.sync_copy(w_s.at[s, k], wv)
            for j in range(4):
                pltpu.async_copy(xs.at[slab].at[riv.at[j]], gbuf.at[j], sem).wait()

            def scale(i16, _):
                base = i16 * 16
                for j in range(4):
                    wvec = wv[j, pl.ds(base, 16)]
                    for u in range(16):
                        wsc = wvec[u]
                        e = base + u
                        gbuf[j, e, pl.ds(0, 16)] = gbuf[j, e, pl.ds(0, 16)] * wsc
                        gbuf[j, e, pl.ds(16, 16)] = gbuf[j, e, pl.ds(16, 16)] * wsc
                return 0
            lax.fori_loop(0, 8, scale, 0)
            for j in range(4):
                pltpu.sync_copy(gbuf.at[j], acc.at[civ.at[j]], add=True)
            return 0
        lax.fori_loop(0, NCK, echunk, 0)
        plsc.subcore_barrier()

        def wb(i, _):
            sl = pl.ds(s * RPT + i * 496, 496)
            pltpu.sync_copy(acc.at[sl], out.at[slab].at[sl])
            return 0
        lax.fori_loop(0, RPT // 496, wb, 0)
        plsc.subcore_barrier()


@functools.cache
def _sc_kernels():
    mesh = plsc.VectorSubcoreMesh(core_axis_name="c", subcore_axis_name="s",
                                  num_cores=2, num_subcores=NSUB)
    sc_params = pltpu.CompilerParams(needs_layout_passes=False,
                                     use_tc_tiling_on_sc=False)
    edge_weight_kernel = pl.kernel(
        _edge_weight_body,
        out_type=jax.ShapeDtypeStruct((32, NUCK, UCKR, 128), jnp.float32),
        mesh=mesh,
        compiler_params=sc_params,
        scratch_types=[
            pltpu.VMEM((UCKR, 128), jnp.int32),
            pltpu.VMEM((UCKR, 128), jnp.float32),
            pltpu.VMEM_SHARED((NN,), jnp.float32),
            pltpu.VMEM((NN,), jnp.float32),
            pltpu.VMEM((UCKR, 128), jnp.int32),
            pltpu.VMEM((UCKR, 128), jnp.int32),
            pltpu.VMEM((UCKR, 128), jnp.float32),
            pltpu.VMEM((UCKR, 128), jnp.float32),
            pltpu.VMEM((496,), jnp.float32),
        ],
    )
    spmm_kernel = pl.kernel(
        _spmm_body,
        out_type=jax.ShapeDtypeStruct((NSLAB, NN, CSLAB), jnp.float32),
        mesh=mesh,
        compiler_params=sc_params,
        scratch_types=[
            pltpu.VMEM((4, 128), jnp.int32),
            pltpu.VMEM((4, 128), jnp.int32),
            pltpu.VMEM((4, 128), jnp.float32),
            pltpu.VMEM((4, 128, CSLAB), jnp.float32),
            pltpu.VMEM((496, CSLAB), jnp.float32),
            pltpu.VMEM_SHARED((NN, CSLAB), jnp.float32),
            pltpu.SemaphoreType.DMA,
        ],
    )
    return edge_weight_kernel, spmm_kernel


def _comb1_body(x_ref, t1_ref, s1_ref, g_ref, b_ref, o_ref):
    cat = jnp.concatenate([x_ref[0], t1_ref[0], s1_ref[0]], axis=1)
    r = jnp.dot(cat, g_ref[...], preferred_element_type=jnp.float32) + b_ref[0]
    o_ref[0] = jnp.where(r > 0, r, 0.01 * r)


def _comb2_body(x_ref, t1_ref, s1_ref, g_ref, b_ref, o_ref):
    cat = jnp.concatenate([x_ref[0], t1_ref[0], s1_ref[0]], axis=1)
    r = jnp.dot(cat, g_ref[...], preferred_element_type=jnp.float32) + b_ref[0]
    o_ref[0] = r


def _gru_body(xc_ref, wih_ref, whh_ref, bih_ref, bhh_ref, wlin_ref, blin_ref,
              o_ref, h_ref):
    t = pl.program_id(0)

    @pl.when(t == 0)
    def _():
        h_ref[...] = jnp.zeros((BBG, HH), jnp.float32)

    h = h_ref[...]
    gi = jnp.dot(xc_ref[0], wih_ref[0],
                 preferred_element_type=jnp.float32) + bih_ref[0]
    gh = jnp.dot(h, whh_ref[...],
                 preferred_element_type=jnp.float32) + bhh_ref[0]
    r = jax.nn.sigmoid(gi[:, :HH] + gh[:, :HH])
    z = jax.nn.sigmoid(gi[:, HH:2 * HH] + gh[:, HH:2 * HH])
    n = jnp.tanh(gi[:, 2 * HH:] + r * gh[:, 2 * HH:])
    h = (1.0 - z) * n + z * h
    h_ref[...] = h

    @pl.when(t == TT - 1)
    def _():
        o_ref[...] = jnp.dot(h, wlin_ref[...],
                             preferred_element_type=jnp.float32) + blin_ref[0]


def _combine(xa, xb, xc_, g, b, body, out_shape, out_spec):
    blk = 992
    grid = (NSLAB, NN // blk)
    in_spec = pl.BlockSpec((1, blk, CSLAB), lambda si, gi: (si, gi, 0))
    w_spec = pl.BlockSpec(g.shape, lambda si, gi: (0,) * g.ndim)
    b_spec = pl.BlockSpec(b.shape, lambda si, gi: (0,) * b.ndim)
    return pl.pallas_call(
        body,
        grid=grid,
        in_specs=[in_spec, in_spec, in_spec, w_spec, b_spec],
        out_specs=out_spec,
        out_shape=out_shape,
    )(xa, xb, xc_, g, b)


def _cheb_mix(w3, b):
    eye2 = jnp.eye(2, dtype=jnp.float32)
    d0 = jnp.kron(eye2, w3[0])
    d1 = jnp.kron(eye2, w3[1])
    d2 = jnp.kron(eye2, w3[2])
    g = jnp.concatenate([d0 - d2, d1, 2.0 * d2], axis=0)
    bb = jnp.tile(b, 2)[None, :]
    return g, bb


def kernel(x, edge_index, edge_weight, batch, W1, b1, W2, b2,
           Wih, Whh, bih, bhh, Wlin, blin):
    row = edge_index[0]
    col = edge_index[1]

    row_u = row.reshape(32, NUCK, UCKR, 128)
    col_u = col.reshape(32, NUCK, UCKR, 128)
    ew_u = edge_weight.reshape(32, NUCK, UCKR, 128)
    row_s = row.reshape(NSUB, NCK, 4, 128)
    col_s = col.reshape(NSUB, NCK, 4, 128)

    _edge_weight_kernel, _spmm_kernel = _sc_kernels()
    w = _edge_weight_kernel(row_u, col_u, ew_u)
    w_s = w.reshape(NSUB, NCK, 4, 128)

    x0 = (x.transpose(2, 0, 1)
           .reshape(NSLAB, 2, NN, FF)
           .transpose(0, 2, 1, 3)
           .reshape(NSLAB, NN, CSLAB))

    tx1 = _spmm_kernel(x0, row_s, col_s, w_s)
    sx1 = _spmm_kernel(tx1, row_s, col_s, w_s)

    g1, b1b = _cheb_mix(W1, b1)
    h1 = _combine(
        x0, tx1, sx1, g1, b1b, _comb1_body,
        jax.ShapeDtypeStruct((NSLAB, NN, CSLAB), jnp.float32),
        pl.BlockSpec((1, 992, CSLAB), lambda si, gi: (si, gi, 0)))

    ty1 = _spmm_kernel(h1, row_s, col_s, w_s)
    sy1 = _spmm_kernel(ty1, row_s, col_s, w_s)

    g2, b2b = _cheb_mix(W2, b2)
    h2 = _combine(
        h1, ty1, sy1, g2, b2b, _comb2_body,
        jax.ShapeDtypeStruct((NSLAB, NN, CSLAB), jnp.float32),
        pl.BlockSpec((1, 992, CSLAB), lambda si, gi: (si, gi, 0)))

    h2r = h2.reshape(NSLAB, BBG, NCHG * CSLAB)
    wv = Wih.T.reshape(NCHG, FF, 3 * HH)
    zpad = jnp.zeros((NCHG, FF, 3 * HH), jnp.float32)
    wbig = jnp.stack([
        jnp.concatenate([wv, zpad], axis=1).reshape(NCHG * CSLAB, 3 * HH),
        jnp.concatenate([zpad, wv], axis=1).reshape(NCHG * CSLAB, 3 * HH),
    ])

    whh_t = Whh.T
    wlin_t = Wlin.T
    out = pl.pallas_call(
        _gru_body,
        grid=(TT,),
        in_specs=[
            pl.BlockSpec((1, BBG, NCHG * CSLAB), lambda t: (t // 2, 0, 0)),
            pl.BlockSpec((1, NCHG * CSLAB, 3 * HH), lambda t: (t % 2, 0, 0)),
            pl.BlockSpec(whh_t.shape, lambda t: (0, 0)),
            pl.BlockSpec((1, 3 * HH), lambda t: (0, 0)),
            pl.BlockSpec((1, 3 * HH), lambda t: (0, 0)),
            pl.BlockSpec(wlin_t.shape, lambda t: (0, 0)),
            pl.BlockSpec((1, NOUT), lambda t: (0, 0)),
        ],
        out_specs=pl.BlockSpec((BBG, NOUT), lambda t: (0, 0)),
        out_shape=jax.ShapeDtypeStruct((BBG, NOUT), jnp.float32),
        scratch_shapes=[pltpu.VMEM((BBG, HH), jnp.float32)],
    )(h2r, wbig, whh_t, bih[None, :], bhh[None, :], wlin_t, blin[None, :])
    return out
